# baseline (speedup 1.0000x reference)
import functools

import jax
import jax.numpy as jnp
from jax import lax
from jax.experimental import pallas as pl
from jax.experimental.pallas import tpu as pltpu
from jax.experimental.pallas import tpu_sc as plsc

BATCH = 16384
DIM = 32
NSEM = 4


def kernel(user_idx, item_idx, user_emb, item_emb):
    info = plsc.get_sparse_core_info()
    nw = info.num_cores * info.num_subcores
    b_per_w = BATCH // nw

    uidx = user_idx.astype(jnp.int32)
    iidx = item_idx.astype(jnp.int32)

    mesh = plsc.VectorSubcoreMesh(core_axis_name="c", subcore_axis_name="s")

    @functools.partial(
        pl.kernel,
        mesh=mesh,
        out_type=(
            jax.ShapeDtypeStruct((BATCH, DIM), jnp.float32),
            jax.ShapeDtypeStruct((BATCH, DIM), jnp.float32),
        ),
        scratch_types=[
            pltpu.VMEM((b_per_w,), jnp.int32),
            pltpu.VMEM((b_per_w,), jnp.int32),
            pltpu.VMEM((b_per_w // 2, DIM), jnp.float32),
            pltpu.VMEM((b_per_w // 2, DIM), jnp.float32),
            [pltpu.SemaphoreType.DMA] * NSEM,
            pltpu.SemaphoreType.DMA,
        ],
    )
    def mf_gather(uidx_hbm, iidx_hbm, uemb_hbm, iemb_hbm, out_u, out_i,
                  uidx_v, iidx_v, urows, irows, gsems, osem):
        wid = lax.axis_index("s") * info.num_cores + lax.axis_index("c")
        base = wid * b_per_w
        pltpu.sync_copy(uidx_hbm.at[pl.ds(base, b_per_w)], uidx_v)
        pltpu.sync_copy(iidx_hbm.at[pl.ds(base, b_per_w)], iidx_v)

        half = b_per_w // 2
        for c in range(2):
            @plsc.parallel_loop(0, half // 16)
            def issue(j):
                uvec = uidx_v[pl.ds(c * half + j * 16, 16)]
                ivec = iidx_v[pl.ds(c * half + j * 16, 16)]
                for l in range(16):
                    pltpu.make_async_copy(
                        uemb_hbm.at[uvec[l]],
                        urows.at[j * 16 + l], gsems[l % NSEM]).start()
                    pltpu.make_async_copy(
                        iemb_hbm.at[ivec[l]],
                        irows.at[j * 16 + l], gsems[l % NSEM]).start()

            for s in range(NSEM):
                pltpu.make_async_copy(
                    uemb_hbm.at[pl.ds(0, half // NSEM)],
                    urows.at[pl.ds(s * (half // NSEM), half // NSEM)],
                    gsems[s]).wait()
                pltpu.make_async_copy(
                    iemb_hbm.at[pl.ds(0, half // NSEM)],
                    irows.at[pl.ds(s * (half // NSEM), half // NSEM)],
                    gsems[s]).wait()

            ou = pltpu.make_async_copy(
                urows, out_u.at[pl.ds(base + c * half, half)], osem)
            oi = pltpu.make_async_copy(
                irows, out_i.at[pl.ds(base + c * half, half)], osem)
            ou.start()
            oi.start()
            ou.wait()
            oi.wait()

    return mf_gather(uidx, iidx, user_emb, item_emb)
